# baseline (speedup 1.0000x reference)
import functools

import jax
import jax.numpy as jnp
from jax import lax
from jax.experimental import pallas as pl
from jax.experimental.pallas import tpu as pltpu
from jax.experimental.pallas import tpu_sc as plsc


def _argmin_body(n_table, tn, nsteps, codesm2_ref, tabA_ref, tabB_ref,
                 t2A_ref, t2B_ref, q2_ref, idx_ref, runmin_ref, runstep_ref):
    step = pl.program_id(0)

    b = runmin_ref.shape[0]

    @pl.when(step == 0)
    def _init():
        runmin_ref[...] = jnp.full((b, tn), jnp.inf, jnp.float32)
        runstep_ref[...] = jnp.zeros((b, tn), jnp.int16)

    def _dist(tab_ref, t2_ref):
        d = lax.dot_general(codesm2_ref[...], tab_ref[...],
                            (((1,), (1,)), ((), ())),
                            preferred_element_type=jnp.float32)
        return (d + q2_ref[...][:, None]) + t2_ref[0]

    distA = _dist(tabA_ref, t2A_ref)
    distB = _dist(tabB_ref, t2B_ref)
    useB = distB < distA
    combined = jnp.minimum(distA, distB)
    two_s = 2 * step
    stepcode = jnp.where(useB, two_s + 1, two_s).astype(jnp.int16)
    better = combined < runmin_ref[...]
    runstep_ref[...] = jnp.where(better, stepcode, runstep_ref[...])
    runmin_ref[...] = jnp.minimum(combined, runmin_ref[...])

    @pl.when(step == nsteps - 1)
    def _final():
        merged = runmin_ref[...]
        winstep = runstep_ref[...]
        lane = lax.broadcasted_iota(jnp.int32, merged.shape, 1)
        rowmin = jnp.min(merged, axis=1)
        gidx = winstep.astype(jnp.int32) * tn + lane
        cand = jnp.where(merged == rowmin[:, None], gidx, jnp.int32(2**30))
        idx_ref[...] = jnp.min(cand, axis=1)


def _nearest_idx(codes_m2, codes_table, t2, q2):
    b, cl = codes_m2.shape
    n = codes_table.shape[0]
    tn = 2000
    halves = n // tn
    steps = halves // 2
    return pl.pallas_call(
        functools.partial(_argmin_body, n, tn, steps),
        grid=(steps,),
        in_specs=[
            pl.BlockSpec((b, cl), lambda i: (0, 0)),
            pl.BlockSpec((tn, cl), lambda i: (2 * i, 0)),
            pl.BlockSpec((tn, cl), lambda i: (2 * i + 1, 0)),
            pl.BlockSpec((1, 1, tn), lambda i: (2 * i, 0, 0)),
            pl.BlockSpec((1, 1, tn), lambda i: (2 * i + 1, 0, 0)),
            pl.BlockSpec((b,), lambda i: (0,)),
        ],
        out_specs=pl.BlockSpec((b,), lambda i: (0,)),
        out_shape=jax.ShapeDtypeStruct((b,), jnp.int32),
        scratch_shapes=[pltpu.VMEM((b, tn), jnp.float32),
                        pltpu.VMEM((b, tn), jnp.int16)],
    )(codes_m2, codes_table, codes_table,
      t2.reshape(halves, 1, tn), t2.reshape(halves, 1, tn), q2)


def _sc_gather(codes_table, idx):
    b = idx.shape[0]
    d = codes_table.shape[1]
    info = plsc.get_sparse_core_info()
    nc, ns = info.num_cores, info.num_subcores
    nw = nc * ns
    b_per_w = b // nw
    mesh = plsc.VectorSubcoreMesh(core_axis_name="c", subcore_axis_name="s")

    @functools.partial(
        pl.kernel, mesh=mesh,
        out_type=jax.ShapeDtypeStruct((b, d), jnp.float32),
        scratch_types=[
            pltpu.VMEM((b_per_w,), jnp.int32),
            pltpu.VMEM((b_per_w, d), jnp.float32),
            pltpu.SemaphoreType.DMA,
        ],
    )
    def gather_kernel(table_hbm, idx_hbm, out_hbm, idx_v, rows_v, sem):
        wid = lax.axis_index("s") * nc + lax.axis_index("c")
        base = wid * b_per_w
        pltpu.sync_copy(idx_hbm.at[pl.ds(base, b_per_w)], idx_v)
        pltpu.async_copy(table_hbm.at[idx_v], rows_v, sem).wait()
        pltpu.sync_copy(rows_v, out_hbm.at[pl.ds(base, b_per_w)])

    return gather_kernel(codes_table, idx)


def _mlp_body(pc_ref, ilW, ilb, h1W, h1b, s2W, s2b, s3W, s3b, h2W, h2b,
              o1W, o1b, o2W, o2b, h3W, h3b, muW, mub, sW, sb,
              mu_ref, ls_ref):
    def dot(a, w):
        return lax.dot_general(a, w[...], (((1,), (0,)), ((), ())),
                               preferred_element_type=jnp.float32)

    x = pc_ref[...]
    i = jnp.tanh(dot(x, ilW) + ilb[...])
    _h1 = jnp.tanh(dot(i, h1W) + h1b[...])
    _s2 = jnp.tanh(dot(_h1, s2W) + s2b[...])
    _s3 = jnp.tanh(dot(_h1, s3W) + s3b[...])
    _h2 = jnp.tanh(dot(_h1 + _s2, h2W) + h2b[...])
    _o1 = jnp.tanh(dot(_h1, o1W) + o1b[...])
    _o2 = jnp.tanh(dot(_h2, o2W) + o2b[...])
    _o3 = jnp.tanh(dot(_h2 + _s3, h3W) + h3b[...])
    out = _o1 + _o2 + _o3
    mu_ref[...] = dot(out, muW) + mub[...]
    ls_ref[...] = dot(out, sW) + sb[...]


def _mlp(prev_code, params):
    b, cl = prev_code.shape
    nh = params[0].shape[1]
    bm = 256
    w_spec = lambda fi, fo: pl.BlockSpec((fi, fo), lambda i: (0, 0))
    b_spec = lambda fo: pl.BlockSpec((fo,), lambda i: (0,))
    in_specs = [pl.BlockSpec((bm, cl), lambda i: (i, 0))]
    for p in params:
        if p.ndim == 2:
            in_specs.append(w_spec(*p.shape))
        else:
            in_specs.append(b_spec(p.shape[0]))
    return pl.pallas_call(
        _mlp_body,
        grid=(b // bm,),
        in_specs=in_specs,
        out_specs=[pl.BlockSpec((bm, cl), lambda i: (i, 0)),
                   pl.BlockSpec((bm, cl), lambda i: (i, 0))],
        out_shape=[jax.ShapeDtypeStruct((b, cl), jnp.float32),
                   jax.ShapeDtypeStruct((b, cl), jnp.float32)],
    )(prev_code, *params)


def kernel(codes, codes_table, input_layer_W, input_layer_b, skipin_to_2_W,
           skipin_to_2_b, skipin_to_3_W, skipin_to_3_b, skip1_to_out_W,
           skip1_to_out_b, skip2_to_out_W, skip2_to_out_b, h1_W, h1_b, h2_W,
           h2_b, h3_W, h3_b, fc_mu_W, fc_mu_b, fc_s_W, fc_s_b):
    q2 = jnp.sum(jnp.square(codes), axis=1)
    t2 = jnp.sum(jnp.square(codes_table), axis=1)
    idx = _nearest_idx(-2.0 * codes, codes_table, t2, q2)
    prev_code = _sc_gather(codes_table, idx)
    params = (input_layer_W, input_layer_b, h1_W, h1_b, skipin_to_2_W,
              skipin_to_2_b, skipin_to_3_W, skipin_to_3_b, h2_W, h2_b,
              skip1_to_out_W, skip1_to_out_b, skip2_to_out_W, skip2_to_out_b,
              h3_W, h3_b, fc_mu_W, fc_mu_b, fc_s_W, fc_s_b)
    mu, logstd = _mlp(prev_code, params)
    return (mu, logstd)


# speedup vs baseline: 1.1942x; 1.1942x over previous
import functools

import jax
import jax.numpy as jnp
from jax import lax
from jax.experimental import pallas as pl
from jax.experimental.pallas import tpu as pltpu
from jax.experimental.pallas import tpu_sc as plsc


def _argmin_body(n_table, tn, nsteps, codesm2_ref, tab_ref, t2_ref, q2_ref,
                 idx_ref, runmin_ref, runstep_ref):
    step = pl.program_id(0)

    b = runmin_ref.shape[0]

    @pl.when(step == 0)
    def _init():
        runmin_ref[...] = jnp.full((b, tn), jnp.inf, jnp.float32)
        runstep_ref[...] = jnp.zeros((b, tn), jnp.int16)

    d = lax.dot_general(codesm2_ref[...], tab_ref[...],
                        (((1,), (1,)), ((), ())),
                        preferred_element_type=jnp.float32)
    dist = (d + q2_ref[...][:, None]) + t2_ref[0]
    better = dist < runmin_ref[...]
    runstep_ref[...] = jnp.where(better, step.astype(jnp.int16), runstep_ref[...])
    runmin_ref[...] = jnp.minimum(dist, runmin_ref[...])

    @pl.when(step == nsteps - 1)
    def _final():
        merged = runmin_ref[...]
        winstep = runstep_ref[...]
        lane = lax.broadcasted_iota(jnp.int32, merged.shape, 1)
        rowmin = jnp.min(merged, axis=1)
        gidx = winstep.astype(jnp.int32) * tn + lane
        cand = jnp.where(merged == rowmin[:, None], gidx, jnp.int32(2**30))
        idx_ref[...] = jnp.min(cand, axis=1)


def _nearest_idx(codes_m2, codes_table, t2, q2):
    b, cl = codes_m2.shape
    n = codes_table.shape[0]
    tn = 2000
    steps = n // tn
    return pl.pallas_call(
        functools.partial(_argmin_body, n, tn, steps),
        grid=(steps,),
        in_specs=[
            pl.BlockSpec((b, cl), lambda i: (0, 0)),
            pl.BlockSpec((tn, cl), lambda i: (i, 0)),
            pl.BlockSpec((1, 1, tn), lambda i: (i, 0, 0)),
            pl.BlockSpec((b,), lambda i: (0,)),
        ],
        out_specs=pl.BlockSpec((b,), lambda i: (0,)),
        out_shape=jax.ShapeDtypeStruct((b,), jnp.int32),
        scratch_shapes=[pltpu.VMEM((b, tn), jnp.float32),
                        pltpu.VMEM((b, tn), jnp.int16)],
    )(codes_m2, codes_table, t2.reshape(steps, 1, tn), q2)


def _sc_gather(codes_table, idx):
    b = idx.shape[0]
    d = codes_table.shape[1]
    info = plsc.get_sparse_core_info()
    nc, ns = info.num_cores, info.num_subcores
    nw = nc * ns
    b_per_w = b // nw
    mesh = plsc.VectorSubcoreMesh(core_axis_name="c", subcore_axis_name="s")

    @functools.partial(
        pl.kernel, mesh=mesh,
        out_type=jax.ShapeDtypeStruct((b, d), jnp.float32),
        scratch_types=[
            pltpu.VMEM((b_per_w,), jnp.int32),
            pltpu.VMEM((b_per_w, d), jnp.float32),
            pltpu.SemaphoreType.DMA,
        ],
    )
    def gather_kernel(table_hbm, idx_hbm, out_hbm, idx_v, rows_v, sem):
        wid = lax.axis_index("s") * nc + lax.axis_index("c")
        base = wid * b_per_w
        pltpu.sync_copy(idx_hbm.at[pl.ds(base, b_per_w)], idx_v)
        pltpu.async_copy(table_hbm.at[idx_v], rows_v, sem).wait()
        pltpu.sync_copy(rows_v, out_hbm.at[pl.ds(base, b_per_w)])

    return gather_kernel(codes_table, idx)


def _mlp_body(pc_ref, ilW, ilb, h1W, h1b, s2W, s2b, s3W, s3b, h2W, h2b,
              o1W, o1b, o2W, o2b, h3W, h3b, muW, mub, sW, sb,
              mu_ref, ls_ref):
    def dot(a, w):
        return lax.dot_general(a, w[...], (((1,), (0,)), ((), ())),
                               preferred_element_type=jnp.float32)

    x = pc_ref[...]
    i = jnp.tanh(dot(x, ilW) + ilb[...])
    _h1 = jnp.tanh(dot(i, h1W) + h1b[...])
    _s2 = jnp.tanh(dot(_h1, s2W) + s2b[...])
    _s3 = jnp.tanh(dot(_h1, s3W) + s3b[...])
    _h2 = jnp.tanh(dot(_h1 + _s2, h2W) + h2b[...])
    _o1 = jnp.tanh(dot(_h1, o1W) + o1b[...])
    _o2 = jnp.tanh(dot(_h2, o2W) + o2b[...])
    _o3 = jnp.tanh(dot(_h2 + _s3, h3W) + h3b[...])
    out = _o1 + _o2 + _o3
    mu_ref[...] = dot(out, muW) + mub[...]
    ls_ref[...] = dot(out, sW) + sb[...]


def _mlp(prev_code, params):
    b, cl = prev_code.shape
    nh = params[0].shape[1]
    bm = 256
    w_spec = lambda fi, fo: pl.BlockSpec((fi, fo), lambda i: (0, 0))
    b_spec = lambda fo: pl.BlockSpec((fo,), lambda i: (0,))
    in_specs = [pl.BlockSpec((bm, cl), lambda i: (i, 0))]
    for p in params:
        if p.ndim == 2:
            in_specs.append(w_spec(*p.shape))
        else:
            in_specs.append(b_spec(p.shape[0]))
    return pl.pallas_call(
        _mlp_body,
        grid=(b // bm,),
        in_specs=in_specs,
        out_specs=[pl.BlockSpec((bm, cl), lambda i: (i, 0)),
                   pl.BlockSpec((bm, cl), lambda i: (i, 0))],
        out_shape=[jax.ShapeDtypeStruct((b, cl), jnp.float32),
                   jax.ShapeDtypeStruct((b, cl), jnp.float32)],
    )(prev_code, *params)


def kernel(codes, codes_table, input_layer_W, input_layer_b, skipin_to_2_W,
           skipin_to_2_b, skipin_to_3_W, skipin_to_3_b, skip1_to_out_W,
           skip1_to_out_b, skip2_to_out_W, skip2_to_out_b, h1_W, h1_b, h2_W,
           h2_b, h3_W, h3_b, fc_mu_W, fc_mu_b, fc_s_W, fc_s_b):
    q2 = jnp.sum(jnp.square(codes), axis=1)
    t2 = jnp.sum(jnp.square(codes_table), axis=1)
    idx = _nearest_idx(-2.0 * codes, codes_table, t2, q2)
    prev_code = _sc_gather(codes_table, idx)
    params = (input_layer_W, input_layer_b, h1_W, h1_b, skipin_to_2_W,
              skipin_to_2_b, skipin_to_3_W, skipin_to_3_b, h2_W, h2_b,
              skip1_to_out_W, skip1_to_out_b, skip2_to_out_W, skip2_to_out_b,
              h3_W, h3_b, fc_mu_W, fc_mu_b, fc_s_W, fc_s_b)
    mu, logstd = _mlp(prev_code, params)
    return (mu, logstd)
